# baseline (speedup 1.0000x reference)
import jax
import jax.numpy as jnp
from jax import lax
from jax.experimental import pallas as pl
from jax.experimental.pallas import tpu as pltpu
from jax.experimental.pallas import tpu_sc as plsc

N = 10000
E = 320000
D = 128

NC = 2
NS = 16
L = 16

C = 128
CPW = 80
HALF = CPW // 2
NW = NC * NS
EP = NW * CPW * C
NBUF = 2

ROWS_A = (N // NS) // 8 * 8
ROWS_LAST = N - (NS - 1) * ROWS_A


def _matmul_body(x_ref, wt_ref, b_ref, h_ref):
    h_ref[...] = (
        jnp.dot(x_ref[...], wt_ref[...], preferred_element_type=jnp.float32)
        + b_ref[...]
    )


def _dense_h(x, wt, b2d):
    grid = 10
    blk = N // grid
    return pl.pallas_call(
        _matmul_body,
        grid=(grid,),
        in_specs=[
            pl.BlockSpec((blk, D), lambda i: (i, 0)),
            pl.BlockSpec((D, D), lambda i: (0, 0)),
            pl.BlockSpec((1, D), lambda i: (0, 0)),
        ],
        out_specs=pl.BlockSpec((blk, D), lambda i: (i, 0)),
        out_shape=jax.ShapeDtypeStruct((N, D), jnp.float32),
    )(x, wt, b2d)


def _add_body(a_ref, b_ref, o_ref):
    o_ref[...] = a_ref[...] + b_ref[...]


def _combine(partials):
    grid = 10
    blk = N // grid
    return pl.pallas_call(
        _add_body,
        grid=(grid,),
        in_specs=[
            pl.BlockSpec((blk, D), lambda i: (i, 0)),
            pl.BlockSpec((blk, D), lambda i: (i + grid, 0)),
        ],
        out_specs=pl.BlockSpec((blk, D), lambda i: (i, 0)),
        out_shape=jax.ShapeDtypeStruct((N, D), jnp.float32),
    )(partials, partials)


def _sc_body(h_hbm, rows_hbm, cols_hbm, vals_hbm, out_hbm,
             cols_v, rows_v, vals_v, rows_buf, acc_sh,
             sg0, sg1, sp0, sp1, sp2, ss0, ss1):
    semg = [sg0, sg1]
    sems = [ss0, ss1]
    c = lax.axis_index("c")
    s = lax.axis_index("s")
    wid = s * NC + c
    start = wid * CPW

    pf0 = pltpu.async_copy(cols_hbm.at[pl.ds(start, HALF)], cols_v, sp0)
    pf1 = pltpu.async_copy(rows_hbm.at[pl.ds(start, HALF)], rows_v, sp1)
    pf2 = pltpu.async_copy(vals_hbm.at[pl.ds(start, HALF)], vals_v, sp2)

    zeros16 = jnp.zeros((L,), jnp.float32)
    zbuf = rows_buf.at[0]

    def _zero_row(r, _):
        for q in range(D // L):
            zbuf[r, pl.ds(q * L, L)] = zeros16
        return 0

    lax.fori_loop(0, C, _zero_row, 0)

    acc_base = s * ROWS_A
    for k in range(ROWS_A // C):
        pltpu.sync_copy(zbuf, acc_sh.at[pl.ds(acc_base + k * C, C)])
    tail0 = ROWS_A - (ROWS_A // C) * C
    pltpu.sync_copy(zbuf.at[pl.ds(0, tail0)],
                    acc_sh.at[pl.ds(acc_base + (ROWS_A // C) * C, tail0)])

    @pl.when(s == NS - 1)
    def _zero_extra():
        extra = ROWS_LAST - ROWS_A
        pltpu.sync_copy(zbuf.at[pl.ds(0, extra)],
                        acc_sh.at[pl.ds(acc_base + ROWS_A, extra)])

    plsc.subcore_barrier()

    def _chunk(j, b):
        nb = 1 - b

        @pl.when(j + 1 < HALF)
        def _issue_next():
            @pl.when(j >= 1)
            def _wait_prev_scatter():
                pltpu.make_async_copy(
                    rows_buf.at[nb], acc_sh.at[rows_v.at[j - 1]], sems[nb]
                ).wait()

            pltpu.async_copy(h_hbm.at[cols_v.at[j + 1]],
                             rows_buf.at[nb], semg[nb])

        pltpu.make_async_copy(h_hbm.at[cols_v.at[j]],
                              rows_buf.at[b], semg[b]).wait()

        rb = rows_buf.at[b]

        def _group(g, _):
            v16 = vals_v[j, pl.ds(g * L, L)]
            for e in range(L):
                r = g * L + e
                bval = jnp.broadcast_to(v16[e], (L,))
                for q in range(D // L):
                    sl = pl.ds(q * L, L)
                    rb[r, sl] = rb[r, sl] * bval
            return 0

        lax.fori_loop(0, C // L, _group, 0)

        pltpu.async_copy(rb, acc_sh.at[rows_v.at[j]], sems[b], add=True)

    for half in range(2):
        if half == 0:
            pf0.wait()
            pf1.wait()
            pf2.wait()
        else:
            base = start + HALF
            pltpu.async_copy(cols_hbm.at[pl.ds(base, HALF)], cols_v, sp0)
            pltpu.async_copy(rows_hbm.at[pl.ds(base, HALF)], rows_v, sp1)
            pltpu.async_copy(vals_hbm.at[pl.ds(base, HALF)], vals_v, sp2)
            pltpu.make_async_copy(
                cols_hbm.at[pl.ds(base, HALF)], cols_v, sp0).wait()
            pltpu.make_async_copy(
                rows_hbm.at[pl.ds(base, HALF)], rows_v, sp1).wait()
            pltpu.make_async_copy(
                vals_hbm.at[pl.ds(base, HALF)], vals_v, sp2).wait()

        pltpu.async_copy(h_hbm.at[cols_v.at[0]], rows_buf.at[0], semg[0])

        def _pair(j2, _):
            for b in range(NBUF):
                _chunk(j2 * NBUF + b, b)
            return 0

        lax.fori_loop(0, HALF // NBUF, _pair, 0)

        for b in range(NBUF):
            pltpu.make_async_copy(
                rows_buf.at[b], acc_sh.at[rows_v.at[HALF - NBUF + b]], sems[b]
            ).wait()

    plsc.subcore_barrier()

    out_base = c * N + acc_base
    for k in range(ROWS_A // C):
        pltpu.sync_copy(acc_sh.at[pl.ds(acc_base + k * C, C)],
                        out_hbm.at[pl.ds(out_base + k * C, C)])
    pltpu.sync_copy(acc_sh.at[pl.ds(acc_base + (ROWS_A // C) * C, tail0)],
                    out_hbm.at[pl.ds(out_base + (ROWS_A // C) * C, tail0)])

    @pl.when(s == NS - 1)
    def _write_extra():
        extra = ROWS_LAST - ROWS_A
        pltpu.sync_copy(acc_sh.at[pl.ds(acc_base + ROWS_A, extra)],
                        out_hbm.at[pl.ds(out_base + ROWS_A, extra)])


def _sc_aggregate(h, rows2d, cols2d, vals2d):
    mesh = plsc.VectorSubcoreMesh(core_axis_name="c", subcore_axis_name="s")
    return pl.kernel(
        _sc_body,
        out_type=jax.ShapeDtypeStruct((NC * N, D), jnp.float32),
        mesh=mesh,
        scratch_types=[
            pltpu.VMEM((HALF, C), jnp.int32),
            pltpu.VMEM((HALF, C), jnp.int32),
            pltpu.VMEM((HALF, C), jnp.float32),
            pltpu.VMEM((NBUF, C, D), jnp.float32),
            pltpu.VMEM_SHARED((N, D), jnp.float32),
        ] + [pltpu.SemaphoreType.DMA] * 7,
    )(h, rows2d, cols2d, vals2d)


def kernel(x, A_indices, A_values, W, b):
    pad = EP - E
    rows = jnp.concatenate([A_indices[0], jnp.zeros((pad,), A_indices.dtype)])
    cols = jnp.concatenate([A_indices[1], jnp.zeros((pad,), A_indices.dtype)])
    vals = jnp.concatenate([A_values, jnp.zeros((pad,), A_values.dtype)])
    rows2d = rows.reshape(EP // C, C)
    cols2d = cols.reshape(EP // C, C)
    vals2d = vals.reshape(EP // C, C)
    h = _dense_h(x, W.T, b.reshape(1, D))
    partials = _sc_aggregate(h, rows2d, cols2d, vals2d)
    return _combine(partials)


# speedup vs baseline: 1.0144x; 1.0144x over previous
import jax
import jax.numpy as jnp
from jax import lax
from jax.experimental import pallas as pl
from jax.experimental.pallas import tpu as pltpu
from jax.experimental.pallas import tpu_sc as plsc

N = 10000
E = 320000
D = 128

NC = 2
NS = 16
L = 16

C = 64
CPW = 160
PART = 16
NPART = CPW // PART
NW = NC * NS
EP = NW * CPW * C
NBUF = 4

ROWS_A = (N // NS) // 8 * 8
ROWS_LAST = N - (NS - 1) * ROWS_A


def _matmul_body(x_ref, wt_ref, b_ref, h_ref):
    h_ref[...] = (
        jnp.dot(x_ref[...], wt_ref[...], preferred_element_type=jnp.float32)
        + b_ref[...]
    )


def _dense_h(x, wt, b2d):
    grid = 10
    blk = N // grid
    return pl.pallas_call(
        _matmul_body,
        grid=(grid,),
        in_specs=[
            pl.BlockSpec((blk, D), lambda i: (i, 0)),
            pl.BlockSpec((D, D), lambda i: (0, 0)),
            pl.BlockSpec((1, D), lambda i: (0, 0)),
        ],
        out_specs=pl.BlockSpec((blk, D), lambda i: (i, 0)),
        out_shape=jax.ShapeDtypeStruct((N, D), jnp.float32),
    )(x, wt, b2d)


def _add_body(a_ref, b_ref, o_ref):
    o_ref[...] = a_ref[...] + b_ref[...]


def _combine(partials):
    grid = 10
    blk = N // grid
    return pl.pallas_call(
        _add_body,
        grid=(grid,),
        in_specs=[
            pl.BlockSpec((blk, D), lambda i: (i, 0)),
            pl.BlockSpec((blk, D), lambda i: (i + grid, 0)),
        ],
        out_specs=pl.BlockSpec((blk, D), lambda i: (i, 0)),
        out_shape=jax.ShapeDtypeStruct((N, D), jnp.float32),
    )(partials, partials)


def _sc_body(h_hbm, rows_hbm, cols_hbm, vals_hbm, out_hbm,
             cols_v, rows_v, vals_v, rows_buf, acc_sh,
             sg0, sg1, sg2, sg3, ss0, ss1, ss2, ss3, sp0, sp1, sp2):
    semg = [sg0, sg1, sg2, sg3]
    sems = [ss0, ss1, ss2, ss3]
    c = lax.axis_index("c")
    s = lax.axis_index("s")
    wid = s * NC + c
    start = wid * CPW

    zeros16 = jnp.zeros((L,), jnp.float32)
    zbuf = rows_buf.at[0]

    def _zero_row(r, _):
        for q in range(D // L):
            zbuf[r, pl.ds(q * L, L)] = zeros16
        return 0

    lax.fori_loop(0, C, _zero_row, 0)

    acc_base = s * ROWS_A
    for k in range(ROWS_A // C):
        pltpu.sync_copy(zbuf, acc_sh.at[pl.ds(acc_base + k * C, C)])
    tail0 = ROWS_A - (ROWS_A // C) * C
    pltpu.sync_copy(zbuf.at[pl.ds(0, tail0)],
                    acc_sh.at[pl.ds(acc_base + (ROWS_A // C) * C, tail0)])

    @pl.when(s == NS - 1)
    def _zero_extra():
        extra = ROWS_LAST - ROWS_A
        pltpu.sync_copy(zbuf.at[pl.ds(0, extra)],
                        acc_sh.at[pl.ds(acc_base + ROWS_A, extra)])

    plsc.subcore_barrier()

    def _wait(sem):
        pltpu.make_async_copy(h_hbm.at[pl.ds(0, C)], rows_buf.at[0], sem
                              ).wait()

    def _wait_idx(sem):
        pltpu.make_async_copy(cols_hbm.at[pl.ds(0, PART)], cols_v, sem
                              ).wait()

    def _scale(j, b):
        rb = rows_buf.at[b]

        def _group(g, _):
            v16 = vals_v[j, pl.ds(g * L, L)]
            for e in range(L):
                r = g * L + e
                bval = jnp.broadcast_to(v16[e], (L,))
                for q in range(D // L):
                    sl = pl.ds(q * L, L)
                    rb[r, sl] = rb[r, sl] * bval
            return 0

        lax.fori_loop(0, C // L, _group, 0)

    def _gather(j, b):
        pltpu.async_copy(h_hbm.at[cols_v.at[j]], rows_buf.at[b], semg[b])

    def _scatter(j, b):
        pltpu.async_copy(rows_buf.at[b], acc_sh.at[rows_v.at[j]],
                         sems[b], add=True)

    def _part(p, _):
        base = start + p * PART
        pltpu.async_copy(cols_hbm.at[pl.ds(base, PART)], cols_v, sp0)
        pltpu.async_copy(rows_hbm.at[pl.ds(base, PART)], rows_v, sp1)
        pltpu.async_copy(vals_hbm.at[pl.ds(base, PART)], vals_v, sp2)
        _wait_idx(sp0)
        _wait_idx(sp1)
        _wait_idx(sp2)

        _gather(0, 0)
        for j in range(NBUF - 1):
            _gather(j + 1, (j + 1) % NBUF)
            _wait(semg[j % NBUF])
            _scale(j, j % NBUF)
            _scatter(j, j % NBUF)

        for t in range((PART - NBUF) // NBUF):
            j0 = (NBUF - 1) + NBUF * t
            for u in range(NBUF):
                j = j0 + u
                b = (NBUF - 1 + u) % NBUF
                nb = (b + 1) % NBUF
                _wait(sems[nb])
                _gather(j + 1, nb)
                _wait(semg[b])
                _scale(j, b)
                _scatter(j, b)

        bl = (PART - 1) % NBUF
        _wait(sems[(bl + 1) % NBUF])
        _wait(semg[bl])
        _scale(PART - 1, bl)
        _scatter(PART - 1, bl)

        for k in range(NBUF - 1):
            _wait(sems[(bl + 2 + k) % NBUF])
        return 0

    lax.fori_loop(0, NPART, _part, 0)

    plsc.subcore_barrier()

    out_base = c * N + acc_base
    for k in range(ROWS_A // C):
        pltpu.sync_copy(acc_sh.at[pl.ds(acc_base + k * C, C)],
                        out_hbm.at[pl.ds(out_base + k * C, C)])
    pltpu.sync_copy(acc_sh.at[pl.ds(acc_base + (ROWS_A // C) * C, tail0)],
                    out_hbm.at[pl.ds(out_base + (ROWS_A // C) * C, tail0)])

    @pl.when(s == NS - 1)
    def _write_extra():
        extra = ROWS_LAST - ROWS_A
        pltpu.sync_copy(acc_sh.at[pl.ds(acc_base + ROWS_A, extra)],
                        out_hbm.at[pl.ds(out_base + ROWS_A, extra)])


def _sc_aggregate(h, rows2d, cols2d, vals2d):
    mesh = plsc.VectorSubcoreMesh(core_axis_name="c", subcore_axis_name="s")
    return pl.kernel(
        _sc_body,
        out_type=jax.ShapeDtypeStruct((NC * N, D), jnp.float32),
        mesh=mesh,
        scratch_types=[
            pltpu.VMEM((PART, C), jnp.int32),
            pltpu.VMEM((PART, C), jnp.int32),
            pltpu.VMEM((PART, C), jnp.float32),
            pltpu.VMEM((NBUF, C, D), jnp.float32),
            pltpu.VMEM_SHARED((N, D), jnp.float32),
        ] + [pltpu.SemaphoreType.DMA] * 11,
    )(h, rows2d, cols2d, vals2d)


def kernel(x, A_indices, A_values, W, b):
    pad = EP - E
    rows = jnp.concatenate([A_indices[0], jnp.zeros((pad,), A_indices.dtype)])
    cols = jnp.concatenate([A_indices[1], jnp.zeros((pad,), A_indices.dtype)])
    vals = jnp.concatenate([A_values, jnp.zeros((pad,), A_values.dtype)])
    rows2d = rows.reshape(EP // C, C)
    cols2d = cols.reshape(EP // C, C)
    vals2d = vals.reshape(EP // C, C)
    h = _dense_h(x, W.T, b.reshape(1, D))
    partials = _sc_aggregate(h, rows2d, cols2d, vals2d)
    return _combine(partials)


# speedup vs baseline: 1.2712x; 1.2532x over previous
import jax
import jax.numpy as jnp
import numpy as np
from jax import lax
from jax.experimental import pallas as pl
from jax.experimental.pallas import tpu as pltpu
from jax.experimental.pallas import tpu_sc as plsc

N = 10000
E = 320000
D = 128

NC = 2
NS = 16
L = 16

C = 64
CPW = 160
PART = 16
NPART = CPW // PART
NW = NC * NS
EP = NW * CPW * C
NG = 3
NSC = 2

ROWS_A = (N // NS) // 8 * 8
ROWS_LAST = N - (NS - 1) * ROWS_A
ZB = 64

_PERM = np.empty(D, np.int64)
for _B in range(D // 32):
    for _i in range(16):
        _PERM[32 * _B + 2 * _i] = 32 * _B + _i
        _PERM[32 * _B + 2 * _i + 1] = 32 * _B + 16 + _i


def _matmul_body(x_ref, wt_ref, b_ref, h_ref):
    h_ref[...] = (
        jnp.dot(x_ref[...], wt_ref[...], preferred_element_type=jnp.float32)
        + b_ref[...]
    ).astype(jnp.bfloat16)


def _dense_h(x, wt, b2d):
    grid = 10
    blk = N // grid
    return pl.pallas_call(
        _matmul_body,
        grid=(grid,),
        in_specs=[
            pl.BlockSpec((blk, D), lambda i: (i, 0)),
            pl.BlockSpec((D, D), lambda i: (0, 0)),
            pl.BlockSpec((1, D), lambda i: (0, 0)),
        ],
        out_specs=pl.BlockSpec((blk, D), lambda i: (i, 0)),
        out_shape=jax.ShapeDtypeStruct((N, D), jnp.bfloat16),
    )(x, wt, b2d)


def _add_body(a_ref, b_ref, o_ref):
    o_ref[...] = a_ref[...] + b_ref[...]


def _combine(partials):
    grid = 10
    blk = N // grid
    return pl.pallas_call(
        _add_body,
        grid=(grid,),
        in_specs=[
            pl.BlockSpec((blk, D), lambda i: (i, 0)),
            pl.BlockSpec((blk, D), lambda i: (i + grid, 0)),
        ],
        out_specs=pl.BlockSpec((blk, D), lambda i: (i, 0)),
        out_shape=jax.ShapeDtypeStruct((N, D), jnp.float32),
    )(partials, partials)


def _sc_body(h_hbm, rows_hbm, cols_hbm, vals_hbm, out_hbm,
             cols_v, rows_v, vals_v, g_ring, s_ring, acc_sh,
             sg0, sg1, sg2, ss0, ss1, sp0, sp1, sp2):
    semg = [sg0, sg1, sg2]
    sems = [ss0, ss1]
    c = lax.axis_index("c")
    s = lax.axis_index("s")
    wid = s * NC + c
    start = wid * CPW

    zeros16 = jnp.zeros((L,), jnp.float32)
    zbuf = s_ring.at[0]

    def _zero_row(r, _):
        for q in range(D // L):
            zbuf[r, pl.ds(q * L, L)] = zeros16
        return 0

    lax.fori_loop(0, ZB, _zero_row, 0)

    acc_base = s * ROWS_A
    for k in range(ROWS_A // ZB):
        pltpu.sync_copy(zbuf, acc_sh.at[pl.ds(acc_base + k * ZB, ZB)])
    tail0 = ROWS_A - (ROWS_A // ZB) * ZB
    pltpu.sync_copy(zbuf.at[pl.ds(0, tail0)],
                    acc_sh.at[pl.ds(acc_base + (ROWS_A // ZB) * ZB, tail0)])

    @pl.when(s == NS - 1)
    def _zero_extra():
        extra = ROWS_LAST - ROWS_A
        pltpu.sync_copy(zbuf.at[pl.ds(0, extra)],
                        acc_sh.at[pl.ds(acc_base + ROWS_A, extra)])

    plsc.subcore_barrier()

    def _wait_g(b):
        pltpu.make_async_copy(h_hbm.at[pl.ds(0, C)], g_ring.at[b], semg[b]
                              ).wait()

    def _wait_s(b):
        pltpu.make_async_copy(out_hbm.at[pl.ds(0, C)], s_ring.at[b], sems[b]
                              ).wait()

    def _wait_idx(sem):
        pltpu.make_async_copy(cols_hbm.at[pl.ds(0, PART)], cols_v, sem
                              ).wait()

    def _scale(j, bg, bs):
        rb = g_ring.at[bg]
        sc = s_ring.at[bs]

        mask_hi = jnp.full((L,), -65536, jnp.int32)

        def _group(g, _):
            v16 = vals_v[j, pl.ds(g * L, L)]
            for e in range(L):
                r = g * L + e
                bval = jnp.broadcast_to(v16[e], (L,))
                for k in range(D // 32):
                    u = rb[r, pl.ds(k * L, L)]
                    lo = plsc.bitcast(lax.shift_left(u, 16), jnp.float32)
                    hi = plsc.bitcast(u & mask_hi, jnp.float32)
                    sc[r, pl.ds(k * 32, L)] = lo * bval
                    sc[r, pl.ds(k * 32 + L, L)] = hi * bval
            return 0

        lax.fori_loop(0, C // L, _group, 0)

    def _gather(j, b):
        pltpu.async_copy(h_hbm.at[cols_v.at[j]], g_ring.at[b], semg[b])

    def _scatter(j, b):
        pltpu.async_copy(s_ring.at[b], acc_sh.at[rows_v.at[j]],
                         sems[b], add=True)

    def _part(p, _):
        base = start + p * PART
        pltpu.async_copy(cols_hbm.at[pl.ds(base, PART)], cols_v, sp0)
        pltpu.async_copy(rows_hbm.at[pl.ds(base, PART)], rows_v, sp1)
        pltpu.async_copy(vals_hbm.at[pl.ds(base, PART)], vals_v, sp2)
        _wait_idx(sp0)
        _wait_idx(sp1)
        _wait_idx(sp2)

        _gather(0, 0)
        _gather(1, 1)
        for j in range(PART):
            if j >= NSC:
                _wait_s(j % NSC)
            if j + 2 < PART:
                _gather(j + 2, (j + 2) % NG)
            _wait_g(j % NG)
            _scale(j, j % NG, j % NSC)
            _scatter(j, j % NSC)

        for k in range(NSC):
            _wait_s((PART - NSC + k) % NSC)
        return 0

    lax.fori_loop(0, NPART, _part, 0)

    plsc.subcore_barrier()

    out_base = c * N + acc_base
    for k in range(ROWS_A // ZB):
        pltpu.sync_copy(acc_sh.at[pl.ds(acc_base + k * ZB, ZB)],
                        out_hbm.at[pl.ds(out_base + k * ZB, ZB)])
    pltpu.sync_copy(acc_sh.at[pl.ds(acc_base + (ROWS_A // ZB) * ZB, tail0)],
                    out_hbm.at[pl.ds(out_base + (ROWS_A // ZB) * ZB, tail0)])

    @pl.when(s == NS - 1)
    def _write_extra():
        extra = ROWS_LAST - ROWS_A
        pltpu.sync_copy(acc_sh.at[pl.ds(acc_base + ROWS_A, extra)],
                        out_hbm.at[pl.ds(out_base + ROWS_A, extra)])


def _sc_aggregate(h, rows2d, cols2d, vals2d):
    mesh = plsc.VectorSubcoreMesh(core_axis_name="c", subcore_axis_name="s")
    return pl.kernel(
        _sc_body,
        out_type=jax.ShapeDtypeStruct((NC * N, D), jnp.float32),
        mesh=mesh,
        compiler_params=pltpu.CompilerParams(needs_layout_passes=False, use_tc_tiling_on_sc=False),
        scratch_types=[
            pltpu.VMEM((PART, C), jnp.int32),
            pltpu.VMEM((PART, C), jnp.int32),
            pltpu.VMEM((PART, C), jnp.float32),
            pltpu.VMEM((NG, C, D // 2), jnp.int32),
            pltpu.VMEM((NSC, C, D), jnp.float32),
            pltpu.VMEM_SHARED((N, D), jnp.float32),
        ] + [pltpu.SemaphoreType.DMA] * 8,
    )(h, rows2d, cols2d, vals2d)


def kernel(x, A_indices, A_values, W, b):
    pad = EP - E
    rows = jnp.concatenate([A_indices[0], jnp.zeros((pad,), A_indices.dtype)])
    cols = jnp.concatenate([A_indices[1], jnp.zeros((pad,), A_indices.dtype)])
    vals = jnp.concatenate([A_values, jnp.zeros((pad,), A_values.dtype)])
    rows2d = rows.reshape(EP // C, C)
    cols2d = cols.reshape(EP // C, C)
    vals2d = vals.reshape(EP // C, C)
    perm = jnp.asarray(_PERM)
    hb = _dense_h(x, W.T[:, perm], b[perm].reshape(1, D))
    h32 = lax.bitcast_convert_type(hb.reshape(N, D // 2, 2), jnp.int32)
    partials = _sc_aggregate(h32, rows2d, cols2d, vals2d)
    return _combine(partials)


# speedup vs baseline: 1.4147x; 1.1129x over previous
import jax
import jax.numpy as jnp
from jax import lax
from jax.experimental import pallas as pl
from jax.experimental.pallas import tpu as pltpu
from jax.experimental.pallas import tpu_sc as plsc

N = 10000
E = 320000
D = 128

NC = 2
NS = 16
L = 16

C = 128
NCHUNK = E // C
NW = NC * NS
CH_BASE = NCHUNK // NW
CH_REM = NCHUNK % NW
ROWS_A = (N // NS) // 8 * 8
ROWS_LAST = N - (NS - 1) * ROWS_A


def _matmul_body(x_ref, wt_ref, b_ref, h_ref):
    h_ref[...] = (
        jnp.dot(x_ref[...], wt_ref[...], preferred_element_type=jnp.float32)
        + b_ref[...]
    )


def _dense_h(x, wt, b2d):
    grid = 10
    blk = N // grid
    return pl.pallas_call(
        _matmul_body,
        grid=(grid,),
        in_specs=[
            pl.BlockSpec((blk, D), lambda i: (i, 0)),
            pl.BlockSpec((D, D), lambda i: (0, 0)),
            pl.BlockSpec((1, D), lambda i: (0, 0)),
        ],
        out_specs=pl.BlockSpec((blk, D), lambda i: (i, 0)),
        out_shape=jax.ShapeDtypeStruct((N, D), jnp.float32),
    )(x, wt, b2d)


def _add_body(a_ref, b_ref, o_ref):
    o_ref[...] = a_ref[...] + b_ref[...]


def _combine(partials):
    grid = 10
    blk = N // grid
    return pl.pallas_call(
        _add_body,
        grid=(grid,),
        in_specs=[
            pl.BlockSpec((blk, D), lambda i: (i, 0)),
            pl.BlockSpec((blk, D), lambda i: (i + grid, 0)),
        ],
        out_specs=pl.BlockSpec((blk, D), lambda i: (i, 0)),
        out_shape=jax.ShapeDtypeStruct((N, D), jnp.float32),
    )(partials, partials)


def _sc_body(h_hbm, rows_hbm, cols_hbm, vals_hbm, out_hbm,
             cols_idx, rows_idx, vals_v, rows_buf, acc_sh, sem):
    c = lax.axis_index("c")
    s = lax.axis_index("s")
    wid = s * NC + c

    zeros16 = jnp.zeros((L,), jnp.float32)

    def _zero_row(r, _):
        for q in range(D // L):
            rows_buf[r, pl.ds(q * L, L)] = zeros16
        return 0

    lax.fori_loop(0, C, _zero_row, 0)

    acc_base = s * ROWS_A

    for k in range(ROWS_A // C):
        pltpu.sync_copy(rows_buf, acc_sh.at[pl.ds(acc_base + k * C, C)])
    tail0 = ROWS_A - (ROWS_A // C) * C
    pltpu.sync_copy(rows_buf.at[pl.ds(0, tail0)],
                    acc_sh.at[pl.ds(acc_base + (ROWS_A // C) * C, tail0)])

    @pl.when(s == NS - 1)
    def _zero_extra():
        extra = ROWS_LAST - ROWS_A
        pltpu.sync_copy(rows_buf.at[pl.ds(0, extra)],
                        acc_sh.at[pl.ds(acc_base + ROWS_A, extra)])

    plsc.subcore_barrier()

    start = wid * CH_BASE + jnp.minimum(wid, CH_REM)
    count = CH_BASE + jnp.where(wid < CH_REM, 1, 0)

    def _chunk(j, _):
        base = (start + j) * C
        pltpu.sync_copy(cols_hbm.at[pl.ds(base, C)], cols_idx)
        pltpu.sync_copy(rows_hbm.at[pl.ds(base, C)], rows_idx)
        pltpu.sync_copy(vals_hbm.at[pl.ds(base, C)], vals_v)
        pltpu.async_copy(h_hbm.at[cols_idx], rows_buf, sem).wait()

        def _group(g, _):
            v16 = vals_v[pl.ds(g * L, L)]
            for e in range(L):
                r = g * L + e
                bval = jnp.broadcast_to(v16[e], (L,))
                for q in range(D // L):
                    sl = pl.ds(q * L, L)
                    rows_buf[r, sl] = rows_buf[r, sl] * bval
            return 0

        lax.fori_loop(0, C // L, _group, 0)

        pltpu.sync_copy(rows_buf, acc_sh.at[rows_idx], add=True)
        return 0

    lax.fori_loop(0, count, _chunk, 0)
    plsc.subcore_barrier()

    out_base = c * N + acc_base
    for k in range(ROWS_A // C):
        pltpu.sync_copy(acc_sh.at[pl.ds(acc_base + k * C, C)],
                        out_hbm.at[pl.ds(out_base + k * C, C)])
    pltpu.sync_copy(acc_sh.at[pl.ds(acc_base + (ROWS_A // C) * C, tail0)],
                    out_hbm.at[pl.ds(out_base + (ROWS_A // C) * C, tail0)])

    @pl.when(s == NS - 1)
    def _write_extra():
        extra = ROWS_LAST - ROWS_A
        pltpu.sync_copy(acc_sh.at[pl.ds(acc_base + ROWS_A, extra)],
                        out_hbm.at[pl.ds(out_base + ROWS_A, extra)])


def _sc_aggregate(h, rows, cols, vals):
    mesh = plsc.VectorSubcoreMesh(core_axis_name="c", subcore_axis_name="s")
    return pl.kernel(
        _sc_body,
        out_type=jax.ShapeDtypeStruct((NC * N, D), jnp.float32),
        mesh=mesh,
        scratch_types=[
            pltpu.VMEM((C,), jnp.int32),
            pltpu.VMEM((C,), jnp.int32),
            pltpu.VMEM((C,), jnp.float32),
            pltpu.VMEM((C, D), jnp.float32),
            pltpu.VMEM_SHARED((N, D), jnp.float32),
            pltpu.SemaphoreType.DMA,
        ],
    )(h, rows, cols, vals)


def kernel(x, A_indices, A_values, W, b):
    rows = A_indices[0]
    cols = A_indices[1]
    h = _dense_h(x, W.T, b.reshape(1, D))
    partials = _sc_aggregate(h, rows, cols, A_values)
    return _combine(partials)
